# Initial kernel scaffold; baseline (speedup 1.0000x reference)
import jax
import jax.numpy as jnp
from jax.experimental import pallas as pl


def kernel(gradients, patch_boxes, transform_decisions):
    raise NotImplementedError("write your pallas kernel here")


# speedup vs baseline: 2.9936x; 2.9936x over previous
import functools

import jax
import jax.numpy as jnp
from jax import lax
from jax.experimental import pallas as pl
from jax.experimental.pallas import tpu as pltpu
from jax.experimental.pallas import tpu_sc as plsc

OH = OW = 100
NT = 104
XPAD = 304
NCROP = 128
NW = 32
CROPS_PER_W = NCROP // NW
NCHUNK = NT // 8
IMLEN = 2 * XPAD + 2 * NT
FMLEN = XPAD + NT * 16
ACC = NT * XPAD


def _build_meta(patch_boxes, B, H, W):
    P = patch_boxes.shape[1]
    boxes = patch_boxes.astype(jnp.float32).reshape(NCROP, 4)
    ymin, xmin, ph, pw = boxes[:, 0], boxes[:, 1], boxes[:, 2], boxes[:, 3]
    b = jnp.repeat(jnp.arange(B, dtype=jnp.float32), P)
    iy = jnp.arange(OH, dtype=jnp.float32) + 0.5
    ix = jnp.arange(OW, dtype=jnp.float32) + 0.5
    rel_y = iy[None, :] * ph[:, None] / OH - 0.5
    rel_x = ix[None, :] * pw[:, None] / OW - 0.5
    y0f = jnp.floor(rel_y)
    x0f = jnp.floor(rel_x)
    wy = rel_y - y0f
    wx = rel_x - x0f
    y0 = jnp.clip(y0f, 0.0, ph[:, None] - 1.0)
    y1 = jnp.clip(y0f + 1.0, 0.0, ph[:, None] - 1.0)
    x0 = jnp.clip(x0f, 0.0, pw[:, None] - 1.0)
    x1 = jnp.clip(x0f + 1.0, 0.0, pw[:, None] - 1.0)
    ay0 = jnp.clip(ymin[:, None] + y0, 0, H - 1)
    ay1 = jnp.clip(ymin[:, None] + y1, 0, H - 1)
    ax0 = jnp.clip(xmin[:, None] + x0, 0, W - 1).astype(jnp.int32)
    ax1 = jnp.clip(xmin[:, None] + x1, 0, W - 1).astype(jnp.int32)
    r0 = (b[:, None] * H + ay0).astype(jnp.int32)
    r1 = (b[:, None] * H + ay1).astype(jnp.int32)
    rows = jnp.zeros((NCROP, NT, 2), jnp.int32)
    rows = rows.at[:, :OH, 0].set(r0).at[:, :OH, 1].set(r1)
    rowids = rows.reshape(NCROP, 2 * NT)
    c3 = jnp.arange(3, dtype=jnp.int32)
    xi0 = jnp.zeros((NCROP, XPAD), jnp.int32)
    xi1 = jnp.zeros((NCROP, XPAD), jnp.int32)
    wxv = jnp.zeros((NCROP, XPAD), jnp.float32)
    xi0 = xi0.at[:, :3 * OW].set((ax0[:, :, None] * 3 + c3).reshape(NCROP, 3 * OW))
    xi1 = xi1.at[:, :3 * OW].set((ax1[:, :, None] * 3 + c3).reshape(NCROP, 3 * OW))
    wxv = wxv.at[:, :3 * OW].set(jnp.repeat(wx, 3, axis=1))
    wyv = jnp.zeros((NCROP, NT, 16), jnp.float32).at[:, :OH, :].set(
        wy[:, :, None])
    imeta = jnp.concatenate([xi0, xi1, rowids], axis=1)
    fmeta = jnp.concatenate([wxv, wyv.reshape(NCROP, NT * 16)], axis=1)
    return imeta, fmeta


@functools.cache
def _sc_accumulate_fn():
    return pl.kernel(
        _sc_accumulate_body,
        out_type=jax.ShapeDtypeStruct((NW, ACC), jnp.float32),
        mesh=plsc.VectorSubcoreMesh(core_axis_name="c", subcore_axis_name="s"),
        compiler_params=pltpu.CompilerParams(needs_layout_passes=False),
        scratch_types=[
            pltpu.VMEM((IMLEN,), jnp.int32),
            pltpu.VMEM((FMLEN,), jnp.float32),
            pltpu.VMEM((16, 1536), jnp.float32),
            pltpu.VMEM((ACC,), jnp.float32),
            pltpu.SemaphoreType.DMA,
        ],
    )


def _sc_accumulate_body(gview, imeta, fmeta, out, imeta_v, fmeta_v, rows_v,
                        acc_v, sem):
    wid = lax.axis_index("s") * 2 + lax.axis_index("c")
    zeros16 = jnp.zeros((16,), jnp.float32)

    def zbody(i, carry):
        acc_v[pl.ds(i * 16, 16)] = zeros16
        return carry

    lax.fori_loop(0, ACC // 16, zbody, 0)

    s0s = [jnp.full((16,), 2 * tt, jnp.int32) for tt in range(8)]
    s1s = [jnp.full((16,), 2 * tt + 1, jnp.int32) for tt in range(8)]

    def crop_body(ci, carry):
        crop = wid * CROPS_PER_W + ci
        pltpu.sync_copy(imeta.at[crop], imeta_v)
        pltpu.sync_copy(fmeta.at[crop], fmeta_v)

        def chunk_body(ch, c2):
            idx = imeta_v[pl.ds(2 * XPAD + ch * 16, 16)]
            pltpu.async_copy(gview.at[idx], rows_v, sem).wait()
            wyvs = [
                fmeta_v[pl.ds(XPAD + (ch * 8 + tt) * 16, 16)]
                for tt in range(8)
            ]
            base0 = ch * (8 * XPAD)
            for k in range(XPAD // 16):
                i0 = imeta_v[pl.ds(k * 16, 16)]
                i1 = imeta_v[pl.ds(XPAD + k * 16, 16)]
                wx = fmeta_v[pl.ds(k * 16, 16)]
                for tt in range(8):
                    v00 = plsc.load_gather(rows_v, [s0s[tt], i0])
                    v01 = plsc.load_gather(rows_v, [s0s[tt], i1])
                    v10 = plsc.load_gather(rows_v, [s1s[tt], i0])
                    v11 = plsc.load_gather(rows_v, [s1s[tt], i1])
                    top = v00 + wx * (v01 - v00)
                    bot = v10 + wx * (v11 - v10)
                    val = top + wyvs[tt] * (bot - top)
                    plsc.addupdate(
                        acc_v.at[pl.ds(base0 + tt * XPAD + k * 16, 16)], val)
            return c2

        lax.fori_loop(0, NCHUNK, chunk_body, 0)
        return carry

    lax.fori_loop(0, CROPS_PER_W, crop_body, 0)
    pltpu.sync_copy(acc_v, out.at[wid])


def _tc_reduce(parts):
    def body(x_ref, o_ref):
        o_ref[...] = jnp.sum(x_ref[...], axis=0)

    return pl.pallas_call(
        body,
        out_shape=jax.ShapeDtypeStruct((ACC,), jnp.float32),
    )(parts)


def kernel(gradients, patch_boxes, transform_decisions):
    B, H, W, C = gradients.shape
    gview = gradients.reshape(B * H, W * C)
    imeta, fmeta = _build_meta(patch_boxes, B, H, W)
    parts = _sc_accumulate_fn()(gview, imeta, fmeta)
    total = _tc_reduce(parts)
    return total.reshape(NT, XPAD)[:OH, :3 * OW].reshape(OH, OW, 3)


# speedup vs baseline: 3.2441x; 1.0837x over previous
import functools

import jax
import jax.numpy as jnp
from jax import lax
from jax.experimental import pallas as pl
from jax.experimental.pallas import tpu as pltpu
from jax.experimental.pallas import tpu_sc as plsc

OH = OW = 100
NT = 104
XPAD = 304
NCROP = 128
NW = 32
CROPS_PER_W = NCROP // NW
NCHUNK = NT // 8
IMLEN = 2 * XPAD + 2 * NT
FMLEN = XPAD + NT * 16
ACC = NT * XPAD


def _build_meta(patch_boxes, B, H, W):
    P = patch_boxes.shape[1]
    boxes = patch_boxes.astype(jnp.float32).reshape(NCROP, 4)
    ymin, xmin, ph, pw = boxes[:, 0], boxes[:, 1], boxes[:, 2], boxes[:, 3]
    b = jnp.repeat(jnp.arange(B, dtype=jnp.float32), P)
    iy = jnp.arange(OH, dtype=jnp.float32) + 0.5
    ix = jnp.arange(OW, dtype=jnp.float32) + 0.5
    rel_y = iy[None, :] * ph[:, None] / OH - 0.5
    rel_x = ix[None, :] * pw[:, None] / OW - 0.5
    y0f = jnp.floor(rel_y)
    x0f = jnp.floor(rel_x)
    wy = rel_y - y0f
    wx = rel_x - x0f
    y0 = jnp.clip(y0f, 0.0, ph[:, None] - 1.0)
    y1 = jnp.clip(y0f + 1.0, 0.0, ph[:, None] - 1.0)
    x0 = jnp.clip(x0f, 0.0, pw[:, None] - 1.0)
    x1 = jnp.clip(x0f + 1.0, 0.0, pw[:, None] - 1.0)
    ay0 = jnp.clip(ymin[:, None] + y0, 0, H - 1)
    ay1 = jnp.clip(ymin[:, None] + y1, 0, H - 1)
    ax0 = jnp.clip(xmin[:, None] + x0, 0, W - 1).astype(jnp.int32)
    ax1 = jnp.clip(xmin[:, None] + x1, 0, W - 1).astype(jnp.int32)
    r0 = (b[:, None] * H + ay0).astype(jnp.int32)
    r1 = (b[:, None] * H + ay1).astype(jnp.int32)
    rows = jnp.zeros((NCROP, NT, 2), jnp.int32)
    rows = rows.at[:, :OH, 0].set(r0).at[:, :OH, 1].set(r1)
    rowids = rows.reshape(NCROP, 2 * NT)
    c3 = jnp.arange(3, dtype=jnp.int32)
    xi0 = jnp.zeros((NCROP, XPAD), jnp.int32)
    xi1 = jnp.zeros((NCROP, XPAD), jnp.int32)
    wxv = jnp.zeros((NCROP, XPAD), jnp.float32)
    xi0 = xi0.at[:, :3 * OW].set((ax0[:, :, None] * 3 + c3).reshape(NCROP, 3 * OW))
    xi1 = xi1.at[:, :3 * OW].set((ax1[:, :, None] * 3 + c3).reshape(NCROP, 3 * OW))
    wxv = wxv.at[:, :3 * OW].set(jnp.repeat(wx, 3, axis=1))
    wyv = jnp.zeros((NCROP, NT, 16), jnp.float32).at[:, :OH, :].set(
        wy[:, :, None])
    imeta = jnp.concatenate([xi0, xi1, rowids], axis=1)
    fmeta = jnp.concatenate([wxv, wyv.reshape(NCROP, NT * 16)], axis=1)
    return imeta, fmeta


@functools.cache
def _sc_accumulate_fn():
    return pl.kernel(
        _sc_accumulate_body,
        out_type=jax.ShapeDtypeStruct((NW, ACC), jnp.float32),
        mesh=plsc.VectorSubcoreMesh(core_axis_name="c", subcore_axis_name="s"),
        compiler_params=pltpu.CompilerParams(needs_layout_passes=False),
        scratch_types=[
            pltpu.VMEM((IMLEN,), jnp.int32),
            pltpu.VMEM((FMLEN,), jnp.float32),
            pltpu.VMEM((2, 16, 1536), jnp.float32),
            pltpu.VMEM((ACC,), jnp.float32),
            pltpu.SemaphoreType.DMA,
            pltpu.SemaphoreType.DMA,
        ],
    )


def _sc_accumulate_body(gview, imeta, fmeta, out, imeta_v, fmeta_v, rows_v,
                        acc_v, sem_a, sem_b):
    wid = lax.axis_index("s") * 2 + lax.axis_index("c")
    sems = [sem_a, sem_b]
    zeros16 = jnp.zeros((16,), jnp.float32)

    def zbody(i, carry):
        acc_v[pl.ds(i * 16, 16)] = zeros16
        return carry

    lax.fori_loop(0, ACC // 16, zbody, 0)

    s0s = [jnp.full((16,), 2 * tt, jnp.int32) for tt in range(8)]
    s1s = [jnp.full((16,), 2 * tt + 1, jnp.int32) for tt in range(8)]

    def row_idx(ch):
        return imeta_v[pl.ds(2 * XPAD + ch * 16, 16)]

    def compute(ch, buf):
        rows = rows_v.at[buf]
        wyvs = [
            fmeta_v[pl.ds(XPAD + (ch * 8 + tt) * 16, 16)] for tt in range(8)
        ]
        base0 = ch * (8 * XPAD)
        for k in range(XPAD // 16):
            i0 = imeta_v[pl.ds(k * 16, 16)]
            i1 = imeta_v[pl.ds(XPAD + k * 16, 16)]
            wx = fmeta_v[pl.ds(k * 16, 16)]
            for tt in range(8):
                v00 = plsc.load_gather(rows, [s0s[tt], i0])
                v01 = plsc.load_gather(rows, [s0s[tt], i1])
                v10 = plsc.load_gather(rows, [s1s[tt], i0])
                v11 = plsc.load_gather(rows, [s1s[tt], i1])
                top = v00 + wx * (v01 - v00)
                bot = v10 + wx * (v11 - v10)
                val = top + wyvs[tt] * (bot - top)
                plsc.addupdate(
                    acc_v.at[pl.ds(base0 + tt * XPAD + k * 16, 16)], val)

    def crop_body(ci, carry):
        crop = wid * CROPS_PER_W + ci
        pltpu.sync_copy(imeta.at[crop], imeta_v)
        pltpu.sync_copy(fmeta.at[crop], fmeta_v)
        pltpu.async_copy(gview.at[row_idx(0)], rows_v.at[0], sems[0])

        def pair_body(ch2, c2):
            for b in range(2):
                ch = ch2 * 2 + b
                pltpu.make_async_copy(
                    gview.at[row_idx(ch)], rows_v.at[b], sems[b]).wait()
                pltpu.async_copy(
                    gview.at[row_idx(ch + 1)], rows_v.at[1 - b],
                    sems[1 - b])
                compute(ch, b)
            return c2

        lax.fori_loop(0, (NCHUNK - 1) // 2, pair_body, 0)
        pltpu.make_async_copy(
            gview.at[row_idx(NCHUNK - 1)], rows_v.at[0],
            sems[0]).wait()
        compute(NCHUNK - 1, 0)
        return carry

    lax.fori_loop(0, CROPS_PER_W, crop_body, 0)
    pltpu.sync_copy(acc_v, out.at[wid])


def _tc_reduce(parts):
    def body(x_ref, o_ref):
        o_ref[...] = jnp.sum(x_ref[...], axis=0)

    return pl.pallas_call(
        body,
        out_shape=jax.ShapeDtypeStruct((ACC,), jnp.float32),
    )(parts)


def kernel(gradients, patch_boxes, transform_decisions):
    B, H, W, C = gradients.shape
    gview = gradients.reshape(B * H, W * C)
    imeta, fmeta = _build_meta(patch_boxes, B, H, W)
    parts = _sc_accumulate_fn()(gview, imeta, fmeta)
    total = _tc_reduce(parts)
    return total.reshape(NT, XPAD)[:OH, :3 * OW].reshape(OH, OW, 3)
